# baseline (speedup 1.0000x reference)
import functools

import jax
import jax.numpy as jnp
from jax import lax
from jax.experimental import pallas as pl
from jax.experimental.pallas import tpu as pltpu
from jax.experimental.pallas import tpu_sc as plsc

N = 8192
D = 1024
FF = 2048
ND = 4
NA = N // ND * 3
NNEWS = N // ND

_NC = 2
_NS = 16
_NW = _NC * _NS
_CHUNK = 64

_G_RPW = NA // _NW
_S_RPW = N // _NW
_NEWS_W = NNEWS // _S_RPW


def _sc_gather_body(x_hbm, idx_hbm, out_hbm, idx_v, rows_v, sem):
    wid = lax.axis_index("s") * _NC + lax.axis_index("c")
    base = wid * _G_RPW
    for c in range(_G_RPW // _CHUNK):
        off = base + c * _CHUNK
        pltpu.sync_copy(idx_hbm.at[pl.ds(off, _CHUNK)], idx_v)
        pltpu.async_copy(x_hbm.at[idx_v], rows_v, sem).wait()
        pltpu.sync_copy(rows_v, out_hbm.at[pl.ds(off, _CHUNK)])


def _sc_gather(x, idx_a):
    return pl.kernel(
        _sc_gather_body,
        out_type=jax.ShapeDtypeStruct((NA, D), jnp.float32),
        mesh=plsc.VectorSubcoreMesh(core_axis_name="c", subcore_axis_name="s"),
        scratch_types=[
            pltpu.VMEM((_CHUNK,), jnp.int32),
            pltpu.VMEM((_CHUNK, D), jnp.float32),
            pltpu.SemaphoreType.DMA,
        ],
    )(x, idx_a)


def _sc_scatter_body(x_hbm, y_hbm, idx_hbm, out_hbm, idx_v, rows_v, sem):
    wid = lax.axis_index("s") * _NC + lax.axis_index("c")
    base = wid * _S_RPW

    @pl.when(wid < _NEWS_W)
    def _news():
        for c in range(_S_RPW // _CHUNK):
            off = base + c * _CHUNK
            pltpu.sync_copy(idx_hbm.at[pl.ds(off, _CHUNK)], idx_v)
            pltpu.async_copy(x_hbm.at[idx_v], rows_v, sem).wait()
            pltpu.async_copy(rows_v, out_hbm.at[idx_v], sem).wait()

    @pl.when(wid >= _NEWS_W)
    def _adapter():
        for c in range(_S_RPW // _CHUNK):
            off = base + c * _CHUNK
            pltpu.sync_copy(idx_hbm.at[pl.ds(off, _CHUNK)], idx_v)
            pltpu.sync_copy(y_hbm.at[pl.ds(off - NNEWS, _CHUNK)], rows_v)
            pltpu.async_copy(rows_v, out_hbm.at[idx_v], sem).wait()


def _sc_scatter(x, y, idx):
    return pl.kernel(
        _sc_scatter_body,
        out_type=jax.ShapeDtypeStruct((N, D), jnp.float32),
        mesh=plsc.VectorSubcoreMesh(core_axis_name="c", subcore_axis_name="s"),
        scratch_types=[
            pltpu.VMEM((_CHUNK,), jnp.int32),
            pltpu.VMEM((_CHUNK, D), jnp.float32),
            pltpu.SemaphoreType.DMA,
        ],
    )(x, y, idx)


_TR = 256
_NT = (N // ND) // _TR


def _tc_adapter_body(
    x_ref, w1_ref, b1_ref, w2_ref, b2_ref, g_ref, b_ref, o_ref, w1b_ref, w2b_ref
):
    @pl.when(pl.program_id(1) == 0)
    def _cast_weights():
        w1b_ref[...] = w1_ref[0].astype(jnp.bfloat16)
        w2b_ref[...] = w2_ref[0].astype(jnp.bfloat16)

    xi = x_ref[...]
    mu = jnp.mean(xi, axis=-1, keepdims=True)
    m2 = jnp.mean(jnp.square(xi), axis=-1, keepdims=True)
    sd = jnp.sqrt(jnp.maximum(m2 - jnp.square(mu), 0.0))
    scale = g_ref[0] / (sd + 1e-6)
    h = (scale * xi + (b_ref[0] - scale * mu)).astype(jnp.bfloat16)
    a = (
        jnp.maximum(
            jnp.dot(
                h,
                w1b_ref[...],
                preferred_element_type=jnp.float32,
            )
            + b1_ref[0],
            0.0,
        )
    ).astype(jnp.bfloat16)
    ff = jnp.dot(
        a,
        w2b_ref[...],
        preferred_element_type=jnp.float32,
    )
    o_ref[...] = (xi + ff) + b2_ref[0]


def _tc_adapter(xg, W1, b1, W2, b2, ln_g, ln_b):
    wmap = lambda d, r: (d, 0, 0)
    return pl.pallas_call(
        _tc_adapter_body,
        grid=(3, _NT),
        in_specs=[
            pl.BlockSpec((_TR, D), lambda d, r: (d * _NT + r, 0)),
            pl.BlockSpec((1, D, FF), wmap),
            pl.BlockSpec((1, 1, FF), wmap),
            pl.BlockSpec((1, FF, D), wmap),
            pl.BlockSpec((1, 1, D), wmap),
            pl.BlockSpec((1, 1, D), wmap),
            pl.BlockSpec((1, 1, D), wmap),
        ],
        out_specs=pl.BlockSpec((_TR, D), lambda d, r: (d * _NT + r, 0)),
        out_shape=jax.ShapeDtypeStruct((NA, D), jnp.float32),
        scratch_shapes=[
            pltpu.VMEM((D, FF), jnp.bfloat16),
            pltpu.VMEM((FF, D), jnp.bfloat16),
        ],
        compiler_params=pltpu.CompilerParams(
            dimension_semantics=("arbitrary", "arbitrary"),
        ),
    )(
        xg,
        W1,
        b1.reshape(3, 1, FF),
        W2,
        b2.reshape(3, 1, D),
        ln_g.reshape(3, 1, D),
        ln_b.reshape(3, 1, D),
    )


def kernel(x, target_domain, W1, b1, W2, b2, ln_g, ln_b):
    idx = target_domain.reshape(N).astype(jnp.int32)
    xg = _sc_gather(x, idx[NNEWS:])
    y = _tc_adapter(xg, W1, b1, W2, b2, ln_g, ln_b)
    out = _sc_scatter(x, y, idx)
    return out


# speedup vs baseline: 1.0587x; 1.0587x over previous
import functools

import jax
import jax.numpy as jnp
from jax import lax
from jax.experimental import pallas as pl
from jax.experimental.pallas import tpu as pltpu
from jax.experimental.pallas import tpu_sc as plsc

N = 8192
D = 1024
FF = 2048
ND = 4
NA = N // ND * 3
NNEWS = N // ND

_NC = 2
_NS = 16
_NW = _NC * _NS
_CHUNK = 64

_G_RPW = NA // _NW
_S_RPW = N // _NW
_NEWS_W = NNEWS // _S_RPW


def _sc_gather_body(x_hbm, idx_hbm, out_hbm, idx_v, rows_v, sem):
    wid = lax.axis_index("s") * _NC + lax.axis_index("c")
    base = wid * _G_RPW
    for c in range(_G_RPW // _CHUNK):
        off = base + c * _CHUNK
        pltpu.sync_copy(idx_hbm.at[pl.ds(off, _CHUNK)], idx_v)
        pltpu.async_copy(x_hbm.at[idx_v], rows_v, sem).wait()
        pltpu.sync_copy(rows_v, out_hbm.at[pl.ds(off, _CHUNK)])


def _sc_gather(x, idx_a):
    return pl.kernel(
        _sc_gather_body,
        out_type=jax.ShapeDtypeStruct((NA, D), jnp.float32),
        mesh=plsc.VectorSubcoreMesh(core_axis_name="c", subcore_axis_name="s"),
        scratch_types=[
            pltpu.VMEM((_CHUNK,), jnp.int32),
            pltpu.VMEM((_CHUNK, D), jnp.float32),
            pltpu.SemaphoreType.DMA,
        ],
    )(x, idx_a)


def _sc_scatter_body(x_hbm, y_hbm, idx_hbm, out_hbm, idx_v, rows_v, sem):
    wid = lax.axis_index("s") * _NC + lax.axis_index("c")
    base = wid * _S_RPW

    @pl.when(wid < _NEWS_W)
    def _news():
        for c in range(_S_RPW // _CHUNK):
            off = base + c * _CHUNK
            pltpu.sync_copy(idx_hbm.at[pl.ds(off, _CHUNK)], idx_v)
            pltpu.async_copy(x_hbm.at[idx_v], rows_v, sem).wait()
            pltpu.async_copy(rows_v, out_hbm.at[idx_v], sem).wait()

    @pl.when(wid >= _NEWS_W)
    def _adapter():
        for c in range(_S_RPW // _CHUNK):
            off = base + c * _CHUNK
            pltpu.sync_copy(idx_hbm.at[pl.ds(off, _CHUNK)], idx_v)
            pltpu.sync_copy(y_hbm.at[pl.ds(off - NNEWS, _CHUNK)], rows_v)
            pltpu.async_copy(rows_v, out_hbm.at[idx_v], sem).wait()


def _sc_scatter(x, y, idx):
    return pl.kernel(
        _sc_scatter_body,
        out_type=jax.ShapeDtypeStruct((N, D), jnp.float32),
        mesh=plsc.VectorSubcoreMesh(core_axis_name="c", subcore_axis_name="s"),
        scratch_types=[
            pltpu.VMEM((_CHUNK,), jnp.int32),
            pltpu.VMEM((_CHUNK, D), jnp.float32),
            pltpu.SemaphoreType.DMA,
        ],
    )(x, y, idx)


_TR = 512
_NT = (N // ND) // _TR


def _tc_adapter_body(x_ref, w1_ref, b1_ref, w2_ref, b2_ref, g_ref, b_ref, o_ref):
    xi = x_ref[...]
    mu = jnp.mean(xi, axis=-1, keepdims=True)
    m2 = jnp.mean(jnp.square(xi), axis=-1, keepdims=True)
    sd = jnp.sqrt(jnp.maximum(m2 - jnp.square(mu), 0.0))
    scale = g_ref[0] / (sd + 1e-6)
    h = (scale * xi + (b_ref[0] - scale * mu)).astype(jnp.bfloat16)
    a = (
        jnp.maximum(
            jnp.dot(
                h,
                w1_ref[0].astype(jnp.bfloat16),
                preferred_element_type=jnp.float32,
            )
            + b1_ref[0],
            0.0,
        )
    ).astype(jnp.bfloat16)
    ff = jnp.dot(
        a,
        w2_ref[0].astype(jnp.bfloat16),
        preferred_element_type=jnp.float32,
    )
    o_ref[...] = (xi + ff) + b2_ref[0]


def _tc_adapter(xg, W1, b1, W2, b2, ln_g, ln_b):
    wmap = lambda d, r: (d, 0, 0)
    return pl.pallas_call(
        _tc_adapter_body,
        grid=(3, _NT),
        in_specs=[
            pl.BlockSpec((_TR, D), lambda d, r: (d * _NT + r, 0)),
            pl.BlockSpec((1, D, FF), wmap),
            pl.BlockSpec((1, 1, FF), wmap),
            pl.BlockSpec((1, FF, D), wmap),
            pl.BlockSpec((1, 1, D), wmap),
            pl.BlockSpec((1, 1, D), wmap),
            pl.BlockSpec((1, 1, D), wmap),
        ],
        out_specs=pl.BlockSpec((_TR, D), lambda d, r: (d * _NT + r, 0)),
        out_shape=jax.ShapeDtypeStruct((NA, D), jnp.float32),
        compiler_params=pltpu.CompilerParams(
            dimension_semantics=("arbitrary", "arbitrary"),
        ),
    )(
        xg,
        W1,
        b1.reshape(3, 1, FF),
        W2,
        b2.reshape(3, 1, D),
        ln_g.reshape(3, 1, D),
        ln_b.reshape(3, 1, D),
    )


def kernel(x, target_domain, W1, b1, W2, b2, ln_g, ln_b):
    idx = target_domain.reshape(N).astype(jnp.int32)
    xg = _sc_gather(x, idx[NNEWS:])
    y = _tc_adapter(xg, W1, b1, W2, b2, ln_g, ln_b)
    out = _sc_scatter(x, y, idx)
    return out


# speedup vs baseline: 1.1051x; 1.0438x over previous
import functools

import jax
import jax.numpy as jnp
from jax import lax
from jax.experimental import pallas as pl
from jax.experimental.pallas import tpu as pltpu
from jax.experimental.pallas import tpu_sc as plsc

N = 8192
D = 1024
FF = 2048
ND = 4
NA = N // ND * 3
NNEWS = N // ND

_NC = 2
_NS = 16
_NW = _NC * _NS
_CHUNK = 64

_G_RPW = NA // _NW
_S_RPW = N // _NW
_NEWS_W = NNEWS // _S_RPW


def _sc_gather_body(x_hbm, idx_hbm, out_hbm, idx_v, rows_v, sem):
    wid = lax.axis_index("s") * _NC + lax.axis_index("c")
    base = wid * _G_RPW
    for c in range(_G_RPW // _CHUNK):
        off = base + c * _CHUNK
        pltpu.sync_copy(idx_hbm.at[pl.ds(off, _CHUNK)], idx_v)
        pltpu.async_copy(x_hbm.at[idx_v], rows_v, sem).wait()
        pltpu.sync_copy(rows_v, out_hbm.at[pl.ds(off, _CHUNK)])


def _sc_gather(x, idx_a):
    return pl.kernel(
        _sc_gather_body,
        out_type=jax.ShapeDtypeStruct((NA, D), jnp.float32),
        mesh=plsc.VectorSubcoreMesh(core_axis_name="c", subcore_axis_name="s"),
        scratch_types=[
            pltpu.VMEM((_CHUNK,), jnp.int32),
            pltpu.VMEM((_CHUNK, D), jnp.float32),
            pltpu.SemaphoreType.DMA,
        ],
    )(x, idx_a)


def _sc_scatter_body(x_hbm, y_hbm, idx_hbm, out_hbm, idx_v, rows_v, sem):
    wid = lax.axis_index("s") * _NC + lax.axis_index("c")
    base = wid * _S_RPW

    @pl.when(wid < _NEWS_W)
    def _news():
        for c in range(_S_RPW // _CHUNK):
            off = base + c * _CHUNK
            pltpu.sync_copy(idx_hbm.at[pl.ds(off, _CHUNK)], idx_v)
            pltpu.async_copy(x_hbm.at[idx_v], rows_v, sem).wait()
            pltpu.async_copy(rows_v, out_hbm.at[idx_v], sem).wait()

    @pl.when(wid >= _NEWS_W)
    def _adapter():
        for c in range(_S_RPW // _CHUNK):
            off = base + c * _CHUNK
            pltpu.sync_copy(idx_hbm.at[pl.ds(off, _CHUNK)], idx_v)
            pltpu.sync_copy(y_hbm.at[pl.ds(off - NNEWS, _CHUNK)], rows_v)
            pltpu.async_copy(rows_v, out_hbm.at[idx_v], sem).wait()


def _sc_scatter(x, y, idx):
    return pl.kernel(
        _sc_scatter_body,
        out_type=jax.ShapeDtypeStruct((N, D), jnp.float32),
        mesh=plsc.VectorSubcoreMesh(core_axis_name="c", subcore_axis_name="s"),
        scratch_types=[
            pltpu.VMEM((_CHUNK,), jnp.int32),
            pltpu.VMEM((_CHUNK, D), jnp.float32),
            pltpu.SemaphoreType.DMA,
        ],
    )(x, y, idx)


_TR = 1024
_NT = (N // ND) // _TR


def _tc_adapter_body(x_ref, w1_ref, b1_ref, w2_ref, b2_ref, g_ref, b_ref, o_ref):
    xi = x_ref[...]
    mu = jnp.mean(xi, axis=-1, keepdims=True)
    m2 = jnp.mean(jnp.square(xi), axis=-1, keepdims=True)
    sd = jnp.sqrt(jnp.maximum(m2 - jnp.square(mu), 0.0))
    scale = g_ref[0] / (sd + 1e-6)
    h = (scale * xi + (b_ref[0] - scale * mu)).astype(jnp.bfloat16)
    a = (
        jnp.maximum(
            jnp.dot(
                h,
                w1_ref[0].astype(jnp.bfloat16),
                preferred_element_type=jnp.float32,
            )
            + b1_ref[0],
            0.0,
        )
    ).astype(jnp.bfloat16)
    ff = jnp.dot(
        a,
        w2_ref[0].astype(jnp.bfloat16),
        preferred_element_type=jnp.float32,
    )
    o_ref[...] = (xi + ff) + b2_ref[0]


def _tc_adapter(xg, W1, b1, W2, b2, ln_g, ln_b):
    wmap = lambda d, r: (d, 0, 0)
    return pl.pallas_call(
        _tc_adapter_body,
        grid=(3, _NT),
        in_specs=[
            pl.BlockSpec((_TR, D), lambda d, r: (d * _NT + r, 0)),
            pl.BlockSpec((1, D, FF), wmap),
            pl.BlockSpec((1, 1, FF), wmap),
            pl.BlockSpec((1, FF, D), wmap),
            pl.BlockSpec((1, 1, D), wmap),
            pl.BlockSpec((1, 1, D), wmap),
            pl.BlockSpec((1, 1, D), wmap),
        ],
        out_specs=pl.BlockSpec((_TR, D), lambda d, r: (d * _NT + r, 0)),
        out_shape=jax.ShapeDtypeStruct((NA, D), jnp.float32),
        compiler_params=pltpu.CompilerParams(
            dimension_semantics=("arbitrary", "arbitrary"),
        ),
    )(
        xg,
        W1,
        b1.reshape(3, 1, FF),
        W2,
        b2.reshape(3, 1, D),
        ln_g.reshape(3, 1, D),
        ln_b.reshape(3, 1, D),
    )


def kernel(x, target_domain, W1, b1, W2, b2, ln_g, ln_b):
    idx = target_domain.reshape(N).astype(jnp.int32)
    xg = _sc_gather(x, idx[NNEWS:])
    y = _tc_adapter(xg, W1, b1, W2, b2, ln_g, ln_b)
    out = _sc_scatter(x, y, idx)
    return out
